# baseline (speedup 1.0000x reference)
import functools

import jax
import jax.numpy as jnp
from jax import lax
from jax.experimental import pallas as pl
from jax.experimental.pallas import tpu as pltpu
from jax.experimental.pallas import tpu_sc as plsc

_NC = 2
_NS = 16
_NW = _NC * _NS

_G = 512
_NI = 8
_NJ = 4
_GB = _G // 128
_P = 129


@functools.partial(jax.jit, static_argnums=(2, 3, 4))
def _sc_lookup(x_t, table, J, I, D):
    jpw = J // _NJ
    fb = D // 8
    mesh = plsc.VectorSubcoreMesh(core_axis_name="c", subcore_axis_name="s")

    @functools.partial(
        pl.kernel,
        mesh=mesh,
        out_type=jax.ShapeDtypeStruct((J, fb, I // 128, 8, 128), jnp.float32),
        scratch_types=[
            pltpu.VMEM((jpw * _G,), jnp.int32),
            pltpu.VMEM((_G, D), jnp.float32),
            pltpu.VMEM((_G, D), jnp.float32),
            pltpu.VMEM((_G, D), jnp.float32),
            pltpu.VMEM((fb, _GB, 8, _P), jnp.float32),
            pltpu.VMEM((fb, _GB, 8, _P), jnp.float32),
            pltpu.SemaphoreType.DMA((3,)),
            pltpu.SemaphoreType.DMA((2,)),
        ],
        compiler_params=pltpu.CompilerParams(
            use_tc_tiling_on_sc=False, needs_layout_passes=False),
    )
    def k(xt_hbm, table_hbm, out_hbm, idx_v, rb0, rb1, rb2, tb0, tb1,
          gsem, wsem):
        wid = lax.axis_index("s") * _NC + lax.axis_index("c")
        ic0 = (wid % _NI) * _GB
        i0 = (wid % _NI) * _G
        jbase = (wid // _NI) * jpw
        rbufs = (rb0, rb1, rb2)
        tbufs = (tb0, tb1)
        lane = jnp.arange(16, dtype=jnp.int32)
        fbv = [(lane + 16 * h) // 8 for h in range(D // 16)]
        f8v = [(lane + 16 * h) % 8 for h in range(D // 16)]
        zero16 = jnp.zeros((16,), jnp.int32)

        def stage(jj, carry):
            pltpu.sync_copy(xt_hbm.at[jbase + jj, pl.ds(i0, _G)],
                            idx_v.at[pl.ds(jj * _G, _G)])
            return carry
        lax.fori_loop(0, jpw, stage, 0)

        def gather_start(jj, b):
            pltpu.make_async_copy(
                table_hbm.at[idx_v.at[pl.ds(jj * _G, _G)]],
                rbufs[b], gsem.at[b]).start()

        def gather_wait(b):
            pltpu.make_async_copy(
                table_hbm.at[idx_v.at[pl.ds(0, _G)]],
                rbufs[b], gsem.at[b]).wait()

        def transpose(src, dst):
            def gbody(g, c):
                icv = zero16 + (g // 128)
                i128v = zero16 + (g % 128)
                for h in range(D // 16):
                    vals = src[g, pl.ds(h * 16, 16)]
                    plsc.store_scatter(dst, [fbv[h], icv, f8v[h], i128v], vals)
                return c
            lax.fori_loop(0, _G, gbody, 0)

        def wb_start(jj, b):
            for f in range(fb):
                pltpu.make_async_copy(
                    tbufs[b].at[f, :, :, pl.ds(0, 128)],
                    out_hbm.at[jbase + jj, f, pl.ds(ic0, _GB), :, :],
                    wsem.at[b]).start()

        def wb_wait(b):
            for f in range(fb):
                pltpu.make_async_copy(
                    tbufs[b].at[f, :, :, pl.ds(0, 128)],
                    out_hbm.at[jbase, f, pl.ds(ic0, _GB), :, :],
                    wsem.at[b]).wait()

        gather_start(0, 0)
        gather_start(1, 1)
        for jj in range(jpw):
            b = jj % 3
            tb = jj & 1
            if jj + 2 < jpw:
                gather_start(jj + 2, (jj + 2) % 3)
            gather_wait(b)
            if jj >= 2:
                wb_wait(tb)
            transpose(rbufs[b], tbufs[tb])
            wb_start(jj, tb)
        wb_wait((jpw - 1) & 1)
        if jpw >= 2:
            wb_wait((jpw - 2) & 1)

    return k(x_t, table)


def kernel(x, table):
    D = table.shape[1]
    I, J = x.shape
    x_t = x.astype(jnp.int32).T
    out5 = _sc_lookup(x_t, table, J, I, D)
    return out5.transpose(2, 4, 0, 1, 3).reshape(I, J, D)


# speedup vs baseline: 1.0778x; 1.0778x over previous
import functools

import jax
import jax.numpy as jnp
from jax import lax
from jax.experimental import pallas as pl
from jax.experimental.pallas import tpu as pltpu
from jax.experimental.pallas import tpu_sc as plsc

_NC = 2
_NS = 16
_NW = _NC * _NS

_G = 512
_NI = 8
_NJ = 4
_GB = _G // 128
_P = 129


@functools.partial(jax.jit, static_argnums=(2, 3, 4))
def _sc_lookup(x_t, table, J, I, D):
    jpw = J // _NJ
    fb = D // 8
    mesh = plsc.VectorSubcoreMesh(core_axis_name="c", subcore_axis_name="s")

    @functools.partial(
        pl.kernel,
        mesh=mesh,
        out_type=jax.ShapeDtypeStruct((J, fb, I // 128, 8, 128), jnp.float32),
        scratch_types=[
            pltpu.VMEM((jpw, _G), jnp.int32),
            pltpu.VMEM((_G, D), jnp.float32),
            pltpu.VMEM((_G, D), jnp.float32),
            pltpu.VMEM((fb, _GB, 8, _P), jnp.float32),
            pltpu.VMEM((fb, _GB, 8, _P), jnp.float32),
            pltpu.SemaphoreType.DMA((2,)),
            pltpu.SemaphoreType.DMA((2,)),
        ],
        compiler_params=pltpu.CompilerParams(
            use_tc_tiling_on_sc=False, needs_layout_passes=False),
    )
    def k(xt_hbm, table_hbm, out_hbm, idx_v, rb0, rb1, tb0, tb1,
          gsem, wsem):
        wid = lax.axis_index("s") * _NC + lax.axis_index("c")
        ic0 = (wid % _NI) * _GB
        i0 = (wid % _NI) * _G
        jbase = (wid // _NI) * jpw
        rbufs = (rb0, rb1)
        tbufs = (tb0, tb1)
        lane = jnp.arange(16, dtype=jnp.int32)
        fbv = [(lane + 16 * h) // 8 for h in range(D // 16)]
        f8v = [(lane + 16 * h) % 8 for h in range(D // 16)]
        zero16 = jnp.zeros((16,), jnp.int32)

        pltpu.sync_copy(xt_hbm.at[pl.ds(jbase, jpw), pl.ds(i0, _G)], idx_v)

        def gather_start(jj, b):
            pltpu.make_async_copy(
                table_hbm.at[idx_v.at[jj]],
                rbufs[b], gsem.at[b]).start()

        def gather_wait(b):
            pltpu.make_async_copy(
                table_hbm.at[idx_v.at[0]],
                rbufs[b], gsem.at[b]).wait()

        def transpose(src, dst):
            for icl in range(_GB):
                icv = zero16 + icl

                def ibody(it, c, icl=icl, icv=icv):
                    base = it * 8
                    for u in range(8):
                        g = icl * 128 + base + u
                        i128v = zero16 + (base + u)
                        for h in range(D // 16):
                            vals = src[g, pl.ds(h * 16, 16)]
                            plsc.store_scatter(
                                dst, [fbv[h], icv, f8v[h], i128v], vals)
                    return c

                lax.fori_loop(0, 16, ibody, 0)

        def wb_start(jj, b):
            for f in range(fb):
                pltpu.make_async_copy(
                    tbufs[b].at[f, :, :, pl.ds(0, 128)],
                    out_hbm.at[jbase + jj, f, pl.ds(ic0, _GB), :, :],
                    wsem.at[b]).start()

        def wb_wait(b):
            for f in range(fb):
                pltpu.make_async_copy(
                    tbufs[b].at[f, :, :, pl.ds(0, 128)],
                    out_hbm.at[jbase, f, pl.ds(ic0, _GB), :, :],
                    wsem.at[b]).wait()

        gather_start(0, 0)

        def jblock(jb, carry):
            for u in range(2):
                jj = jb * 2 + u
                @pl.when(jj + 1 < jpw)
                def _():
                    gather_start(jj + 1, 1 - u)
                gather_wait(u)
                @pl.when(jj >= 2)
                def _():
                    wb_wait(u)
                transpose(rbufs[u], tbufs[u])
                wb_start(jj, u)
            return carry

        lax.fori_loop(0, jpw // 2, jblock, 0)
        wb_wait(0)
        wb_wait(1)

    return k(x_t, table)


def kernel(x, table):
    D = table.shape[1]
    I, J = x.shape
    x_t = x.astype(jnp.int32).T
    out5 = _sc_lookup(x_t, table, J, I, D)
    return out5.transpose(2, 4, 0, 1, 3).reshape(I, J, D)
